# baseline (speedup 1.0000x reference)
import jax
import jax.numpy as jnp
from jax.experimental import pallas as pl
from jax.experimental.pallas import tpu as pltpu


def _poly_w(w, G, J):
    K, Cin, Cout = w.shape
    j = jnp.arange(J)[:, None, None]
    g = jnp.arange(G)[None, :, None]
    r = jnp.arange(G)[None, None, :]
    k = G * j + g - r
    valid = (k >= 0) & (k < K)
    wk = jnp.where(valid[..., None, None], w[jnp.clip(k, 0, K - 1)], 0)
    return jnp.transpose(wk, (0, 1, 3, 2, 4)).reshape(J * G * Cin, G * Cout)


_B = 4


def _conv_feats_kernel(x_ref, w1_ref, b1_ref, w2_ref, b2_ref, w3_ref, b3_ref,
                       w4_ref, b4_ref, w5_ref, b5_ref, w6_ref, b6_ref,
                       nw1_ref, nb1_ref, nw2_ref, nb2_ref, out_ref, pbuf):

    def staggered(items, emit_dot, emit_pool):
        outs, prev = [], None
        for it in items:
            acc = emit_dot(it)
            if prev is not None:
                outs.append(emit_pool(prev))
            prev = acc
        outs.append(emit_pool(prev))
        return outs

    def l1_dot(i):
        acc = jnp.dot(x_ref[i, 0:330, :], w1_ref[0:108, :],
                      preferred_element_type=jnp.float32)
        for j in range(1, 5):
            acc = acc + jnp.dot(x_ref[i, j:j + 330, :],
                                w1_ref[108 * j:108 * (j + 1), :],
                                preferred_element_type=jnp.float32)
        return acc

    def l1_pool(acc):
        yi = jnp.maximum(acc + b1_ref[...], 0.0).astype(jnp.bfloat16)
        return jnp.concatenate(
            [jnp.maximum(jnp.maximum(yi[:, 96 * v:96 * v + 32],
                                     yi[:, 96 * v + 32:96 * v + 64]),
                         yi[:, 96 * v + 64:96 * v + 96]) for v in range(3)],
            axis=1)

    pooled1 = staggered(range(_B), l1_dot, l1_pool)

    def l2_dot(p1):
        acc = jnp.dot(p1[0:327], w2_ref[0:96, :],
                      preferred_element_type=jnp.float32)
        for j in range(1, 4):
            acc = acc + jnp.dot(p1[j:j + 327], w2_ref[96 * j:96 * (j + 1), :],
                                preferred_element_type=jnp.float32)
        return acc

    def l2_pool(acc):
        yi = jnp.maximum(acc + b2_ref[...], 0.0).astype(jnp.bfloat16)
        return jnp.maximum(jnp.maximum(yi[:, 0:64], yi[:, 64:128]),
                           yi[:, 128:192])

    xin = staggered(pooled1, l2_dot, l2_pool)

    h_in = 327
    for w_ref, b_ref, K, Cin, Cout in ((w3_ref, b3_ref, 10, 64, 64),
                                       (w4_ref, b4_ref, 5, 64, 64),
                                       (w5_ref, b5_ref, 5, 64, 128),
                                       (w6_ref, b6_ref, 3, 128, 128)):
        hout = h_in - K + 1
        hp = hout // 3

        def lx_dot(xi, w_ref=w_ref, K=K, Cin=Cin, hout=hout):
            acc = jnp.dot(xi[0:hout], w_ref[0:Cin, :],
                          preferred_element_type=jnp.float32)
            for k in range(1, K):
                acc = acc + jnp.dot(xi[k:k + hout],
                                    w_ref[Cin * k:Cin * (k + 1), :],
                                    preferred_element_type=jnp.float32)
            return acc

        islot = iter(range(_B))

        def lx_pool(acc, b_ref=b_ref, Cout=Cout, hout=hout, hp=hp):
            i = next(islot)
            pbuf[i, pl.ds(0, hout), pl.ds(0, Cout)] = (
                jnp.maximum(acc + b_ref[...], 0.0))
            p0 = pbuf[i, pl.ds(0, hp, stride=3), pl.ds(0, Cout)]
            p1 = pbuf[i, pl.ds(1, hp, stride=3), pl.ds(0, Cout)]
            p2 = pbuf[i, pl.ds(2, hp, stride=3), pl.ds(0, Cout)]
            return jnp.maximum(jnp.maximum(p0, p1), p2).astype(jnp.bfloat16)

        xin = staggered(xin, lx_dot, lx_pool)
        h_in = hp

    f0 = jnp.concatenate([xin[i][0:1] for i in range(_B)], axis=0)
    f1 = jnp.concatenate([xin[i][1:2] for i in range(_B)], axis=0)
    y1 = (jnp.dot(f0, nw1_ref[0], preferred_element_type=jnp.float32)
          + jnp.dot(f1, nw1_ref[1], preferred_element_type=jnp.float32)
          + nb1_ref[...])
    y1 = jnp.maximum(y1, 0.0).astype(jnp.bfloat16)
    y2 = jnp.dot(y1, nw2_ref[...], preferred_element_type=jnp.float32)
    out_ref[:, 0, :] = jnp.maximum(y2 + nb2_ref[...], 0.0).astype(out_ref.dtype)


def _bilstm_head_kernel(feat_ref, wih_ref, bg_ref, whh_ref,
                        w3f_ref, w3b_ref, b3_ref, w4_ref, b4_ref,
                        out_ref, xg_ref):
    T = feat_ref.shape[0]
    Hd = whh_ref.shape[0]
    G4 = 4 * Hd

    xg_ref[...] = (jnp.dot(feat_ref[...], wih_ref[...],
                           preferred_element_type=jnp.float32) + bg_ref[...])

    def cell(s, carry):
        h, c = carry
        hm = jnp.dot(h.astype(jnp.bfloat16), whh_ref[...],
                     preferred_element_type=jnp.float32)
        gf = xg_ref[pl.ds(s, 1), pl.ds(0, G4)] + hm[0:1, 0:G4]
        gr = xg_ref[pl.ds(T - 1 - s, 1), pl.ds(G4, G4)] + hm[1:2, G4:2 * G4]
        g = jnp.concatenate([gf, gr], axis=0)
        i_g = jax.nn.sigmoid(g[:, 0:Hd])
        f_g = jax.nn.sigmoid(g[:, Hd:2 * Hd])
        g_g = jnp.tanh(g[:, 2 * Hd:3 * Hd])
        o_g = jax.nn.sigmoid(g[:, 3 * Hd:4 * Hd])
        c = f_g * c + i_g * g_g
        h = o_g * jnp.tanh(c)
        return h, c

    def step4(q, carry):
        for u in range(4):
            carry = cell(4 * q + u, carry)
        return carry

    z = jnp.zeros((2, Hd), jnp.float32)
    h, _ = jax.lax.fori_loop(0, T // 4, step4, (z, z))

    y3 = (jnp.dot(h[0:1].astype(jnp.bfloat16), w3f_ref[...],
                  preferred_element_type=jnp.float32)
          + jnp.dot(h[1:2].astype(jnp.bfloat16), w3b_ref[...],
                    preferred_element_type=jnp.float32)
          + b3_ref[...])
    out_ref[...] = jnp.dot(y3.astype(jnp.bfloat16), w4_ref[...],
                           preferred_element_type=jnp.float32) + b4_ref[...]


def kernel(x, conv1_w, conv1_b, conv2_w, conv2_b, conv3_w, conv3_b,
           conv4_w, conv4_b, conv5_w, conv5_b, conv6_w, conv6_b,
           n1_w1, n1_b1, n1_w2, n1_b2,
           lstm_wih, lstm_bg, lstm_whh_f, lstm_whh_r,
           n3_w1f, n3_w1b, n3_b1, n3_w2, n3_b2):
    N, C, H, _ = x.shape
    assert (C, H) == (12, 3000), "conv schedule is pinned to C=12, H=3000"

    x_nhc = jnp.transpose(x[..., 0], (0, 2, 1)).astype(jnp.bfloat16)
    SR = 334
    xr = jnp.pad(x_nhc, ((0, 0), (0, 9 * SR - H), (0, 0))).reshape(
        N, SR, 9 * C)

    w1p = _poly_w(conv1_w, 9, 5)
    b1p = jnp.tile(conv1_b, (1, 9))
    w2p = _poly_w(conv2_w, 3, 4)
    b2p = jnp.tile(conv2_b, (1, 3))
    w3 = conv3_w.reshape(-1, conv3_w.shape[2])
    w4 = conv4_w.reshape(-1, conv4_w.shape[2])
    w5 = conv5_w.reshape(-1, conv5_w.shape[2])
    w6 = conv6_w.reshape(-1, conv6_w.shape[2])

    inputs = [xr, w1p, b1p, w2p, b2p, w3, conv3_b, w4, conv4_b, w5, conv5_b,
              w6, conv6_b, n1_w1, n1_b1, n1_w2, n1_b2]
    in_specs = [pl.BlockSpec((_B, SR, 9 * C), lambda n: (n, 0, 0))]
    for a in inputs[1:]:
        in_specs.append(
            pl.BlockSpec(a.shape, lambda n, nd=a.ndim: (0,) * nd))

    feats = pl.pallas_call(
        _conv_feats_kernel,
        out_shape=jax.ShapeDtypeStruct((N, 1, 128), jnp.bfloat16),
        grid_spec=pltpu.PrefetchScalarGridSpec(
            num_scalar_prefetch=0,
            grid=(N // _B,),
            in_specs=in_specs,
            out_specs=pl.BlockSpec((_B, 1, 128), lambda n: (n, 0, 0)),
            scratch_shapes=[pltpu.VMEM((_B, 320, 128), jnp.float32)],
        ),
        compiler_params=pltpu.CompilerParams(
            dimension_semantics=("parallel",),
            vmem_limit_bytes=64 * 1024 * 1024,
        ),
    )(*inputs)

    whh_st = jnp.concatenate([lstm_whh_f, lstm_whh_r], axis=1)
    return pl.pallas_call(
        _bilstm_head_kernel,
        out_shape=jax.ShapeDtypeStruct((1, n3_b2.shape[1]), jnp.float32),
        scratch_shapes=[pltpu.VMEM((N, 2048), jnp.float32)],
    )(feats.reshape(N, 128), lstm_wih, lstm_bg, whh_st,
      n3_w1f, n3_w1b, n3_b1, n3_w2, n3_b2)


# speedup vs baseline: 1.0388x; 1.0388x over previous
import jax
import jax.numpy as jnp
from jax.experimental import pallas as pl
from jax.experimental.pallas import tpu as pltpu


def _poly_w(w, G, J):
    K, Cin, Cout = w.shape
    j = jnp.arange(J)[:, None, None]
    g = jnp.arange(G)[None, :, None]
    r = jnp.arange(G)[None, None, :]
    k = G * j + g - r
    valid = (k >= 0) & (k < K)
    wk = jnp.where(valid[..., None, None], w[jnp.clip(k, 0, K - 1)], 0)
    return jnp.transpose(wk, (0, 1, 3, 2, 4)).reshape(J * G * Cin, G * Cout)


_B = 8


def _conv_feats_kernel(x_ref, w1_ref, b1_ref, w2_ref, b2_ref, w3_ref, b3_ref,
                       w4_ref, b4_ref, w5_ref, b5_ref, w6_ref, b6_ref,
                       nw1_ref, nb1_ref, nw2_ref, nb2_ref, out_ref, pbuf):

    def staggered(items, emit_dot, emit_pool):
        outs, prev = [], None
        for it in items:
            acc = emit_dot(it)
            if prev is not None:
                outs.append(emit_pool(prev))
            prev = acc
        outs.append(emit_pool(prev))
        return outs

    def l1_dot(i):
        acc = jnp.dot(x_ref[i, 0:330, :], w1_ref[0:108, :],
                      preferred_element_type=jnp.float32)
        for j in range(1, 5):
            acc = acc + jnp.dot(x_ref[i, j:j + 330, :],
                                w1_ref[108 * j:108 * (j + 1), :],
                                preferred_element_type=jnp.float32)
        return acc

    def l1_pool(acc):
        yi = jnp.maximum(acc + b1_ref[...], 0.0).astype(jnp.bfloat16)
        return jnp.concatenate(
            [jnp.maximum(jnp.maximum(yi[:, 96 * v:96 * v + 32],
                                     yi[:, 96 * v + 32:96 * v + 64]),
                         yi[:, 96 * v + 64:96 * v + 96]) for v in range(3)],
            axis=1)

    pooled1 = staggered(range(_B), l1_dot, l1_pool)

    def l2_dot(p1):
        acc = jnp.dot(p1[0:327], w2_ref[0:96, :],
                      preferred_element_type=jnp.float32)
        for j in range(1, 4):
            acc = acc + jnp.dot(p1[j:j + 327], w2_ref[96 * j:96 * (j + 1), :],
                                preferred_element_type=jnp.float32)
        return acc

    def l2_pool(acc):
        yi = jnp.maximum(acc + b2_ref[...], 0.0).astype(jnp.bfloat16)
        return jnp.maximum(jnp.maximum(yi[:, 0:64], yi[:, 64:128]),
                           yi[:, 128:192])

    xin = staggered(pooled1, l2_dot, l2_pool)

    h_in = 327
    for w_ref, b_ref, K, Cin, Cout in ((w3_ref, b3_ref, 10, 64, 64),
                                       (w4_ref, b4_ref, 5, 64, 64),
                                       (w5_ref, b5_ref, 5, 64, 128),
                                       (w6_ref, b6_ref, 3, 128, 128)):
        hout = h_in - K + 1
        hp = hout // 3

        def lx_dot(xi, w_ref=w_ref, K=K, Cin=Cin, hout=hout):
            acc = jnp.dot(xi[0:hout], w_ref[0:Cin, :],
                          preferred_element_type=jnp.float32)
            for k in range(1, K):
                acc = acc + jnp.dot(xi[k:k + hout],
                                    w_ref[Cin * k:Cin * (k + 1), :],
                                    preferred_element_type=jnp.float32)
            return acc

        islot = iter(range(_B))

        def lx_pool(acc, b_ref=b_ref, Cout=Cout, hout=hout, hp=hp):
            i = next(islot)
            pbuf[i, pl.ds(0, hout), pl.ds(0, Cout)] = (
                jnp.maximum(acc + b_ref[...], 0.0))
            p0 = pbuf[i, pl.ds(0, hp, stride=3), pl.ds(0, Cout)]
            p1 = pbuf[i, pl.ds(1, hp, stride=3), pl.ds(0, Cout)]
            p2 = pbuf[i, pl.ds(2, hp, stride=3), pl.ds(0, Cout)]
            return jnp.maximum(jnp.maximum(p0, p1), p2).astype(jnp.bfloat16)

        xin = staggered(xin, lx_dot, lx_pool)
        h_in = hp

    f0 = jnp.concatenate([xin[i][0:1] for i in range(_B)], axis=0)
    f1 = jnp.concatenate([xin[i][1:2] for i in range(_B)], axis=0)
    y1 = (jnp.dot(f0, nw1_ref[0], preferred_element_type=jnp.float32)
          + jnp.dot(f1, nw1_ref[1], preferred_element_type=jnp.float32)
          + nb1_ref[...])
    y1 = jnp.maximum(y1, 0.0).astype(jnp.bfloat16)
    y2 = jnp.dot(y1, nw2_ref[...], preferred_element_type=jnp.float32)
    out_ref[:, 0, :] = jnp.maximum(y2 + nb2_ref[...], 0.0).astype(out_ref.dtype)


def _bilstm_head_kernel(feat_ref, wih_ref, bg_ref, whh_ref,
                        w3f_ref, w3b_ref, b3_ref, w4_ref, b4_ref,
                        out_ref, xg_ref):
    T = feat_ref.shape[0]
    Hd = whh_ref.shape[0]
    G4 = 4 * Hd

    xg_ref[...] = (jnp.dot(feat_ref[...], wih_ref[...],
                           preferred_element_type=jnp.float32) + bg_ref[...])

    def cell(s, carry):
        h, c = carry
        hm = jnp.dot(h.astype(jnp.bfloat16), whh_ref[...],
                     preferred_element_type=jnp.float32)
        gf = xg_ref[pl.ds(s, 1), pl.ds(0, G4)] + hm[0:1, 0:G4]
        gr = xg_ref[pl.ds(T - 1 - s, 1), pl.ds(G4, G4)] + hm[1:2, G4:2 * G4]
        g = jnp.concatenate([gf, gr], axis=0)
        i_g = jax.nn.sigmoid(g[:, 0:Hd])
        f_g = jax.nn.sigmoid(g[:, Hd:2 * Hd])
        g_g = jnp.tanh(g[:, 2 * Hd:3 * Hd])
        o_g = jax.nn.sigmoid(g[:, 3 * Hd:4 * Hd])
        c = f_g * c + i_g * g_g
        h = o_g * jnp.tanh(c)
        return h, c

    def step4(q, carry):
        for u in range(4):
            carry = cell(4 * q + u, carry)
        return carry

    z = jnp.zeros((2, Hd), jnp.float32)
    h, _ = jax.lax.fori_loop(0, T // 4, step4, (z, z))

    y3 = (jnp.dot(h[0:1].astype(jnp.bfloat16), w3f_ref[...],
                  preferred_element_type=jnp.float32)
          + jnp.dot(h[1:2].astype(jnp.bfloat16), w3b_ref[...],
                    preferred_element_type=jnp.float32)
          + b3_ref[...])
    out_ref[...] = jnp.dot(y3.astype(jnp.bfloat16), w4_ref[...],
                           preferred_element_type=jnp.float32) + b4_ref[...]


def kernel(x, conv1_w, conv1_b, conv2_w, conv2_b, conv3_w, conv3_b,
           conv4_w, conv4_b, conv5_w, conv5_b, conv6_w, conv6_b,
           n1_w1, n1_b1, n1_w2, n1_b2,
           lstm_wih, lstm_bg, lstm_whh_f, lstm_whh_r,
           n3_w1f, n3_w1b, n3_b1, n3_w2, n3_b2):
    N, C, H, _ = x.shape
    assert (C, H) == (12, 3000), "conv schedule is pinned to C=12, H=3000"

    x_nhc = jnp.transpose(x[..., 0], (0, 2, 1)).astype(jnp.bfloat16)
    SR = 334
    xr = jnp.pad(x_nhc, ((0, 0), (0, 9 * SR - H), (0, 0))).reshape(
        N, SR, 9 * C)

    w1p = _poly_w(conv1_w, 9, 5)
    b1p = jnp.tile(conv1_b, (1, 9))
    w2p = _poly_w(conv2_w, 3, 4)
    b2p = jnp.tile(conv2_b, (1, 3))
    w3 = conv3_w.reshape(-1, conv3_w.shape[2])
    w4 = conv4_w.reshape(-1, conv4_w.shape[2])
    w5 = conv5_w.reshape(-1, conv5_w.shape[2])
    w6 = conv6_w.reshape(-1, conv6_w.shape[2])

    inputs = [xr, w1p, b1p, w2p, b2p, w3, conv3_b, w4, conv4_b, w5, conv5_b,
              w6, conv6_b, n1_w1, n1_b1, n1_w2, n1_b2]
    in_specs = [pl.BlockSpec((_B, SR, 9 * C), lambda n: (n, 0, 0))]
    for a in inputs[1:]:
        in_specs.append(
            pl.BlockSpec(a.shape, lambda n, nd=a.ndim: (0,) * nd))

    feats = pl.pallas_call(
        _conv_feats_kernel,
        out_shape=jax.ShapeDtypeStruct((N, 1, 128), jnp.bfloat16),
        grid_spec=pltpu.PrefetchScalarGridSpec(
            num_scalar_prefetch=0,
            grid=(N // _B,),
            in_specs=in_specs,
            out_specs=pl.BlockSpec((_B, 1, 128), lambda n: (n, 0, 0)),
            scratch_shapes=[pltpu.VMEM((_B, 320, 128), jnp.float32)],
        ),
        compiler_params=pltpu.CompilerParams(
            dimension_semantics=("parallel",),
            vmem_limit_bytes=64 * 1024 * 1024,
        ),
    )(*inputs)

    whh_st = jnp.concatenate([lstm_whh_f, lstm_whh_r], axis=1)
    return pl.pallas_call(
        _bilstm_head_kernel,
        out_shape=jax.ShapeDtypeStruct((1, n3_b2.shape[1]), jnp.float32),
        scratch_shapes=[pltpu.VMEM((N, 2048), jnp.float32)],
    )(feats.reshape(N, 128), lstm_wih, lstm_bg, whh_st,
      n3_w1f, n3_w1b, n3_b1, n3_w2, n3_b2)


# speedup vs baseline: 1.0408x; 1.0019x over previous
import jax
import jax.numpy as jnp
from jax.experimental import pallas as pl
from jax.experimental.pallas import tpu as pltpu


def _poly_w(w, G, J):
    K, Cin, Cout = w.shape
    j = jnp.arange(J)[:, None, None]
    g = jnp.arange(G)[None, :, None]
    r = jnp.arange(G)[None, None, :]
    k = G * j + g - r
    valid = (k >= 0) & (k < K)
    wk = jnp.where(valid[..., None, None], w[jnp.clip(k, 0, K - 1)], 0)
    return jnp.transpose(wk, (0, 1, 3, 2, 4)).reshape(J * G * Cin, G * Cout)


_B = 8


def _conv_feats_kernel(x_ref, w1_ref, b1_ref, w2_ref, b2_ref, w3_ref, b3_ref,
                       w4_ref, b4_ref, w5_ref, b5_ref, w6_ref, b6_ref,
                       nw1_ref, nb1_ref, nw2_ref, nb2_ref, out_ref, pbuf):

    def staggered(items, emit_dot, emit_pool):
        outs, prev = [], None
        for it in items:
            acc = emit_dot(it)
            if prev is not None:
                outs.append(emit_pool(prev))
            prev = acc
        outs.append(emit_pool(prev))
        return outs

    def l1_dot(i):
        acc = jnp.dot(x_ref[i, 0:330, :], w1_ref[0:108, :],
                      preferred_element_type=jnp.float32)
        for j in range(1, 5):
            acc = acc + jnp.dot(x_ref[i, j:j + 330, :],
                                w1_ref[108 * j:108 * (j + 1), :],
                                preferred_element_type=jnp.float32)
        return acc

    def l1_pool(acc):
        yi = jnp.maximum(acc + b1_ref[...], 0.0).astype(jnp.bfloat16)
        return jnp.concatenate(
            [jnp.maximum(jnp.maximum(yi[:, 96 * v:96 * v + 32],
                                     yi[:, 96 * v + 32:96 * v + 64]),
                         yi[:, 96 * v + 64:96 * v + 96]) for v in range(3)],
            axis=1)

    pooled1 = staggered(range(_B), l1_dot, l1_pool)

    def l2_dot(p1):
        acc = jnp.dot(p1[0:327], w2_ref[0:96, :],
                      preferred_element_type=jnp.float32)
        for j in range(1, 4):
            acc = acc + jnp.dot(p1[j:j + 327], w2_ref[96 * j:96 * (j + 1), :],
                                preferred_element_type=jnp.float32)
        return acc

    def l2_pool(acc):
        yi = jnp.maximum(acc + b2_ref[...], 0.0).astype(jnp.bfloat16)
        return jnp.maximum(jnp.maximum(yi[:, 0:64], yi[:, 64:128]),
                           yi[:, 128:192])

    xin = staggered(pooled1, l2_dot, l2_pool)

    h_in = 327
    for w_ref, b_ref, K, Cin, Cout in ((w3_ref, b3_ref, 10, 64, 64),
                                       (w4_ref, b4_ref, 5, 64, 64),
                                       (w5_ref, b5_ref, 5, 64, 128),
                                       (w6_ref, b6_ref, 3, 128, 128)):
        hout = h_in - K + 1
        hp = hout // 3

        def lx_dot(xi, w_ref=w_ref, K=K, Cin=Cin, hout=hout):
            acc = jnp.dot(xi[0:hout], w_ref[0:Cin, :],
                          preferred_element_type=jnp.float32)
            for k in range(1, K):
                acc = acc + jnp.dot(xi[k:k + hout],
                                    w_ref[Cin * k:Cin * (k + 1), :],
                                    preferred_element_type=jnp.float32)
            return acc

        islot = iter(range(_B))

        def lx_pool(acc, b_ref=b_ref, Cout=Cout, hout=hout, hp=hp):
            i = next(islot)
            pbuf[i, pl.ds(0, hout), pl.ds(0, Cout)] = (
                jnp.maximum(acc + b_ref[...], 0.0))
            p0 = pbuf[i, pl.ds(0, hp, stride=3), pl.ds(0, Cout)]
            p1 = pbuf[i, pl.ds(1, hp, stride=3), pl.ds(0, Cout)]
            p2 = pbuf[i, pl.ds(2, hp, stride=3), pl.ds(0, Cout)]
            return jnp.maximum(jnp.maximum(p0, p1), p2).astype(jnp.bfloat16)

        xin = staggered(xin, lx_dot, lx_pool)
        h_in = hp

    f0 = jnp.concatenate([xin[i][0:1] for i in range(_B)], axis=0)
    f1 = jnp.concatenate([xin[i][1:2] for i in range(_B)], axis=0)
    y1 = (jnp.dot(f0, nw1_ref[0], preferred_element_type=jnp.float32)
          + jnp.dot(f1, nw1_ref[1], preferred_element_type=jnp.float32)
          + nb1_ref[...])
    y1 = jnp.maximum(y1, 0.0).astype(jnp.bfloat16)
    y2 = jnp.dot(y1, nw2_ref[...], preferred_element_type=jnp.float32)
    out_ref[:, 0, :] = jnp.maximum(y2 + nb2_ref[...], 0.0).astype(out_ref.dtype)


def _bilstm_head_kernel(feat_ref, wih_ref, bg_ref, whh_ref,
                        w3f_ref, w3b_ref, b3_ref, w4_ref, b4_ref,
                        out_ref, xg_ref):
    T = feat_ref.shape[0]
    Hd = whh_ref.shape[0]
    G4 = 4 * Hd

    xg_ref[...] = (jnp.dot(feat_ref[...], wih_ref[...],
                           preferred_element_type=jnp.float32) + bg_ref[...])

    def cell(s, carry):
        h, c = carry
        hm = jnp.dot(h.astype(jnp.bfloat16), whh_ref[...],
                     preferred_element_type=jnp.float32)
        gf = xg_ref[pl.ds(s, 1), pl.ds(0, G4)] + hm[0:1, 0:G4]
        gr = xg_ref[pl.ds(T - 1 - s, 1), pl.ds(G4, G4)] + hm[1:2, G4:2 * G4]
        g = jnp.concatenate([gf, gr], axis=0)
        i_g = jax.nn.sigmoid(g[:, 0:Hd])
        f_g = jax.nn.sigmoid(g[:, Hd:2 * Hd])
        g_g = jnp.tanh(g[:, 2 * Hd:3 * Hd])
        o_g = jax.nn.sigmoid(g[:, 3 * Hd:4 * Hd])
        c = f_g * c + i_g * g_g
        h = o_g * jnp.tanh(c)
        return h, c

    def step8(q, carry):
        for u in range(8):
            carry = cell(8 * q + u, carry)
        return carry

    z = jnp.zeros((2, Hd), jnp.float32)
    h, _ = jax.lax.fori_loop(0, T // 8, step8, (z, z))

    y3 = (jnp.dot(h[0:1].astype(jnp.bfloat16), w3f_ref[...],
                  preferred_element_type=jnp.float32)
          + jnp.dot(h[1:2].astype(jnp.bfloat16), w3b_ref[...],
                    preferred_element_type=jnp.float32)
          + b3_ref[...])
    out_ref[...] = jnp.dot(y3.astype(jnp.bfloat16), w4_ref[...],
                           preferred_element_type=jnp.float32) + b4_ref[...]


def kernel(x, conv1_w, conv1_b, conv2_w, conv2_b, conv3_w, conv3_b,
           conv4_w, conv4_b, conv5_w, conv5_b, conv6_w, conv6_b,
           n1_w1, n1_b1, n1_w2, n1_b2,
           lstm_wih, lstm_bg, lstm_whh_f, lstm_whh_r,
           n3_w1f, n3_w1b, n3_b1, n3_w2, n3_b2):
    N, C, H, _ = x.shape
    assert (C, H) == (12, 3000), "conv schedule is pinned to C=12, H=3000"

    x_nhc = jnp.transpose(x[..., 0], (0, 2, 1)).astype(jnp.bfloat16)
    SR = 334
    xr = jnp.pad(x_nhc, ((0, 0), (0, 9 * SR - H), (0, 0))).reshape(
        N, SR, 9 * C)

    w1p = _poly_w(conv1_w, 9, 5)
    b1p = jnp.tile(conv1_b, (1, 9))
    w2p = _poly_w(conv2_w, 3, 4)
    b2p = jnp.tile(conv2_b, (1, 3))
    w3 = conv3_w.reshape(-1, conv3_w.shape[2])
    w4 = conv4_w.reshape(-1, conv4_w.shape[2])
    w5 = conv5_w.reshape(-1, conv5_w.shape[2])
    w6 = conv6_w.reshape(-1, conv6_w.shape[2])

    inputs = [xr, w1p, b1p, w2p, b2p, w3, conv3_b, w4, conv4_b, w5, conv5_b,
              w6, conv6_b, n1_w1, n1_b1, n1_w2, n1_b2]
    in_specs = [pl.BlockSpec((_B, SR, 9 * C), lambda n: (n, 0, 0))]
    for a in inputs[1:]:
        in_specs.append(
            pl.BlockSpec(a.shape, lambda n, nd=a.ndim: (0,) * nd))

    feats = pl.pallas_call(
        _conv_feats_kernel,
        out_shape=jax.ShapeDtypeStruct((N, 1, 128), jnp.bfloat16),
        grid_spec=pltpu.PrefetchScalarGridSpec(
            num_scalar_prefetch=0,
            grid=(N // _B,),
            in_specs=in_specs,
            out_specs=pl.BlockSpec((_B, 1, 128), lambda n: (n, 0, 0)),
            scratch_shapes=[pltpu.VMEM((_B, 320, 128), jnp.float32)],
        ),
        compiler_params=pltpu.CompilerParams(
            dimension_semantics=("parallel",),
            vmem_limit_bytes=64 * 1024 * 1024,
        ),
    )(*inputs)

    whh_st = jnp.concatenate([lstm_whh_f, lstm_whh_r], axis=1)
    return pl.pallas_call(
        _bilstm_head_kernel,
        out_shape=jax.ShapeDtypeStruct((1, n3_b2.shape[1]), jnp.float32),
        scratch_shapes=[pltpu.VMEM((N, 2048), jnp.float32)],
    )(feats.reshape(N, 128), lstm_wih, lstm_bg, whh_st,
      n3_w1f, n3_w1b, n3_b1, n3_w2, n3_b2)


# speedup vs baseline: 1.2640x; 1.2145x over previous
import jax
import jax.numpy as jnp
from jax.experimental import pallas as pl
from jax.experimental.pallas import tpu as pltpu


def _poly_w(w, G, J):
    K, Cin, Cout = w.shape
    j = jnp.arange(J)[:, None, None]
    g = jnp.arange(G)[None, :, None]
    r = jnp.arange(G)[None, None, :]
    k = G * j + g - r
    valid = (k >= 0) & (k < K)
    wk = jnp.where(valid[..., None, None], w[jnp.clip(k, 0, K - 1)], 0)
    return jnp.transpose(wk, (0, 1, 3, 2, 4)).reshape(J * G * Cin, G * Cout)


_B = 8


def _conv_feats_kernel(x_ref, w1_ref, b1_ref, w2_ref, b2_ref, w3_ref, b3_ref,
                       w4_ref, b4_ref, w5_ref, b5_ref, w6_ref, b6_ref,
                       nw1_ref, nb1_ref, nw2_ref, nb2_ref, out_ref, pbuf):

    def staggered(items, emit_dot, emit_pool):
        outs, prev = [], None
        for it in items:
            acc = emit_dot(it)
            if prev is not None:
                outs.append(emit_pool(prev))
            prev = acc
        outs.append(emit_pool(prev))
        return outs

    def l1_dot(i):
        acc = None
        for j, nt in ((0, 2), (2, 2), (4, 1)):
            lhs = x_ref[i, j:j + 330, :]
            if nt == 2:
                lhs = jnp.concatenate([lhs, x_ref[i, j + 1:j + 331, :]],
                                      axis=1)
            d = jnp.dot(lhs, w1_ref[108 * j:108 * (j + nt), :],
                        preferred_element_type=jnp.float32)
            acc = d if acc is None else acc + d
        return acc

    def l1_pool(acc):
        yi = jnp.maximum(acc + b1_ref[...], 0.0).astype(jnp.bfloat16)
        return jnp.concatenate(
            [jnp.maximum(jnp.maximum(yi[:, 96 * v:96 * v + 32],
                                     yi[:, 96 * v + 32:96 * v + 64]),
                         yi[:, 96 * v + 64:96 * v + 96]) for v in range(3)],
            axis=1)

    pooled1 = staggered(range(_B), l1_dot, l1_pool)

    def l2_dot(p1):
        acc = None
        for j in (0, 2):
            lhs = jnp.concatenate([p1[j:j + 327], p1[j + 1:j + 328]], axis=1)
            d = jnp.dot(lhs, w2_ref[96 * j:96 * (j + 2), :],
                        preferred_element_type=jnp.float32)
            acc = d if acc is None else acc + d
        return acc

    def l2_pool(acc):
        yi = jnp.maximum(acc + b2_ref[...], 0.0).astype(jnp.bfloat16)
        return jnp.maximum(jnp.maximum(yi[:, 0:64], yi[:, 64:128]),
                           yi[:, 128:192])

    xin = staggered(pooled1, l2_dot, l2_pool)

    h_in = 327
    for w_ref, b_ref, K, Cin, Cout in ((w3_ref, b3_ref, 10, 64, 64),
                                       (w4_ref, b4_ref, 5, 64, 64),
                                       (w5_ref, b5_ref, 5, 64, 128),
                                       (w6_ref, b6_ref, 3, 128, 128)):
        hout = h_in - K + 1
        hp = hout // 3

        def lx_dot(xi, w_ref=w_ref, K=K, Cin=Cin, hout=hout):
            acc = None
            k = 0
            while k < K:
                nt = 2 if (Cin == 64 and k + 1 < K) else 1
                lhs = xi[k:k + hout]
                if nt == 2:
                    lhs = jnp.concatenate([lhs, xi[k + 1:k + 1 + hout]],
                                          axis=1)
                d = jnp.dot(lhs, w_ref[Cin * k:Cin * (k + nt), :],
                            preferred_element_type=jnp.float32)
                acc = d if acc is None else acc + d
                k += nt
            return acc

        islot = iter(range(_B))

        def lx_pool(acc, b_ref=b_ref, Cout=Cout, hout=hout, hp=hp):
            i = next(islot)
            pbuf[i, pl.ds(0, hout), pl.ds(0, Cout)] = (
                jnp.maximum(acc + b_ref[...], 0.0))
            p0 = pbuf[i, pl.ds(0, hp, stride=3), pl.ds(0, Cout)]
            p1 = pbuf[i, pl.ds(1, hp, stride=3), pl.ds(0, Cout)]
            p2 = pbuf[i, pl.ds(2, hp, stride=3), pl.ds(0, Cout)]
            return jnp.maximum(jnp.maximum(p0, p1), p2).astype(jnp.bfloat16)

        xin = staggered(xin, lx_dot, lx_pool)
        h_in = hp

    f0 = jnp.concatenate([xin[i][0:1] for i in range(_B)], axis=0)
    f1 = jnp.concatenate([xin[i][1:2] for i in range(_B)], axis=0)
    y1 = (jnp.dot(f0, nw1_ref[0], preferred_element_type=jnp.float32)
          + jnp.dot(f1, nw1_ref[1], preferred_element_type=jnp.float32)
          + nb1_ref[...])
    y1 = jnp.maximum(y1, 0.0).astype(jnp.bfloat16)
    y2 = jnp.dot(y1, nw2_ref[...], preferred_element_type=jnp.float32)
    out_ref[:, 0, :] = jnp.maximum(y2 + nb2_ref[...], 0.0).astype(out_ref.dtype)


def _bilstm_head_kernel(feat_ref, wih_ref, bg_ref, whh_ref,
                        w3f_ref, w3b_ref, b3_ref, w4_ref, b4_ref,
                        out_ref, xg_ref):
    T = feat_ref.shape[0]
    Hd = whh_ref.shape[0]
    G4 = 4 * Hd

    xg_ref[...] = (jnp.dot(feat_ref[...], wih_ref[...],
                           preferred_element_type=jnp.float32) + bg_ref[...])

    def cell(s, carry):
        h, c = carry
        hm = jnp.dot(h.astype(jnp.bfloat16), whh_ref[...],
                     preferred_element_type=jnp.float32)
        gf = xg_ref[pl.ds(s, 1), pl.ds(0, G4)] + hm[0:1, 0:G4]
        gr = xg_ref[pl.ds(T - 1 - s, 1), pl.ds(G4, G4)] + hm[1:2, G4:2 * G4]
        g = jnp.concatenate([gf, gr], axis=0)
        i_g = jax.nn.sigmoid(g[:, 0:Hd])
        f_g = jax.nn.sigmoid(g[:, Hd:2 * Hd])
        g_g = jnp.tanh(g[:, 2 * Hd:3 * Hd])
        o_g = jax.nn.sigmoid(g[:, 3 * Hd:4 * Hd])
        c = f_g * c + i_g * g_g
        h = o_g * jnp.tanh(c)
        return h, c

    def step8(q, carry):
        for u in range(8):
            carry = cell(8 * q + u, carry)
        return carry

    z = jnp.zeros((2, Hd), jnp.float32)
    h, _ = jax.lax.fori_loop(0, T // 8, step8, (z, z))

    y3 = (jnp.dot(h[0:1].astype(jnp.bfloat16), w3f_ref[...],
                  preferred_element_type=jnp.float32)
          + jnp.dot(h[1:2].astype(jnp.bfloat16), w3b_ref[...],
                    preferred_element_type=jnp.float32)
          + b3_ref[...])
    out_ref[...] = jnp.dot(y3.astype(jnp.bfloat16), w4_ref[...],
                           preferred_element_type=jnp.float32) + b4_ref[...]


def kernel(x, conv1_w, conv1_b, conv2_w, conv2_b, conv3_w, conv3_b,
           conv4_w, conv4_b, conv5_w, conv5_b, conv6_w, conv6_b,
           n1_w1, n1_b1, n1_w2, n1_b2,
           lstm_wih, lstm_bg, lstm_whh_f, lstm_whh_r,
           n3_w1f, n3_w1b, n3_b1, n3_w2, n3_b2):
    N, C, H, _ = x.shape
    assert (C, H) == (12, 3000), "conv schedule is pinned to C=12, H=3000"

    x_nhc = jnp.transpose(x[..., 0], (0, 2, 1)).astype(jnp.bfloat16)
    SR = 334
    xr = jnp.pad(x_nhc, ((0, 0), (0, 9 * SR - H), (0, 0))).reshape(
        N, SR, 9 * C)

    w1p = _poly_w(conv1_w, 9, 5)
    b1p = jnp.tile(conv1_b, (1, 9))
    w2p = _poly_w(conv2_w, 3, 4)
    b2p = jnp.tile(conv2_b, (1, 3))
    w3 = conv3_w.reshape(-1, conv3_w.shape[2])
    w4 = conv4_w.reshape(-1, conv4_w.shape[2])
    w5 = conv5_w.reshape(-1, conv5_w.shape[2])
    w6 = conv6_w.reshape(-1, conv6_w.shape[2])

    inputs = [xr, w1p, b1p, w2p, b2p, w3, conv3_b, w4, conv4_b, w5, conv5_b,
              w6, conv6_b, n1_w1, n1_b1, n1_w2, n1_b2]
    in_specs = [pl.BlockSpec((_B, SR, 9 * C), lambda n: (n, 0, 0))]
    for a in inputs[1:]:
        in_specs.append(
            pl.BlockSpec(a.shape, lambda n, nd=a.ndim: (0,) * nd))

    feats = pl.pallas_call(
        _conv_feats_kernel,
        out_shape=jax.ShapeDtypeStruct((N, 1, 128), jnp.bfloat16),
        grid_spec=pltpu.PrefetchScalarGridSpec(
            num_scalar_prefetch=0,
            grid=(N // _B,),
            in_specs=in_specs,
            out_specs=pl.BlockSpec((_B, 1, 128), lambda n: (n, 0, 0)),
            scratch_shapes=[pltpu.VMEM((_B, 320, 128), jnp.float32)],
        ),
        compiler_params=pltpu.CompilerParams(
            dimension_semantics=("parallel",),
            vmem_limit_bytes=64 * 1024 * 1024,
        ),
    )(*inputs)

    whh_st = jnp.concatenate([lstm_whh_f, lstm_whh_r], axis=1)
    return pl.pallas_call(
        _bilstm_head_kernel,
        out_shape=jax.ShapeDtypeStruct((1, n3_b2.shape[1]), jnp.float32),
        scratch_shapes=[pltpu.VMEM((N, 2048), jnp.float32)],
    )(feats.reshape(N, 128), lstm_wih, lstm_bg, whh_st,
      n3_w1f, n3_w1b, n3_b1, n3_w2, n3_b2)


# speedup vs baseline: 1.2765x; 1.0098x over previous
import jax
import jax.numpy as jnp
from jax.experimental import pallas as pl
from jax.experimental.pallas import tpu as pltpu


def _poly_w(w, G, J):
    K, Cin, Cout = w.shape
    j = jnp.arange(J)[:, None, None]
    g = jnp.arange(G)[None, :, None]
    r = jnp.arange(G)[None, None, :]
    k = G * j + g - r
    valid = (k >= 0) & (k < K)
    wk = jnp.where(valid[..., None, None], w[jnp.clip(k, 0, K - 1)], 0)
    return jnp.transpose(wk, (0, 1, 3, 2, 4)).reshape(J * G * Cin, G * Cout)


_B = 8


def _conv_feats_kernel(x_ref, w1_ref, b1_ref, w2_ref, b2_ref, w3_ref, b3_ref,
                       w4_ref, b4_ref, w5_ref, b5_ref, w6_ref, b6_ref,
                       nw1_ref, nb1_ref, nw2_ref, nb2_ref, out_ref, pbuf):

    def staggered(items, emit_dot, emit_pool):
        outs, prev = [], None
        for it in items:
            acc = emit_dot(it)
            if prev is not None:
                outs.append(emit_pool(prev))
            prev = acc
        outs.append(emit_pool(prev))
        return outs

    def l1_dot(i):
        acc = None
        for j, nt in ((0, 2), (2, 2), (4, 1)):
            lhs = x_ref[i, j:j + 330, :]
            if nt == 2:
                lhs = jnp.concatenate([lhs, x_ref[i, j + 1:j + 331, :]],
                                      axis=1)
            d = jnp.dot(lhs, w1_ref[108 * j:108 * (j + nt), :],
                        preferred_element_type=jnp.float32)
            acc = d if acc is None else acc + d
        return acc

    def l1_pool(acc):
        yi = jnp.maximum(acc + b1_ref[...], 0.0).astype(jnp.bfloat16)
        return jnp.concatenate(
            [jnp.maximum(jnp.maximum(yi[:, 96 * v:96 * v + 32],
                                     yi[:, 96 * v + 32:96 * v + 64]),
                         yi[:, 96 * v + 64:96 * v + 96]) for v in range(3)],
            axis=1)

    pooled1 = staggered(range(_B), l1_dot, l1_pool)

    def l2_dot(p1):
        acc = None
        for j in (0, 2):
            lhs = jnp.concatenate([p1[j:j + 327], p1[j + 1:j + 328]], axis=1)
            d = jnp.dot(lhs, w2_ref[96 * j:96 * (j + 2), :],
                        preferred_element_type=jnp.float32)
            acc = d if acc is None else acc + d
        return acc

    def l2_pool(acc):
        yi = jnp.maximum(acc + b2_ref[...], 0.0).astype(jnp.bfloat16)
        return jnp.maximum(jnp.maximum(yi[:, 0:64], yi[:, 64:128]),
                           yi[:, 128:192])

    xin = staggered(pooled1, l2_dot, l2_pool)

    h_in = 327
    for w_ref, b_ref, K, Cin, Cout in ((w3_ref, b3_ref, 10, 64, 64),
                                       (w4_ref, b4_ref, 5, 64, 64),
                                       (w5_ref, b5_ref, 5, 64, 128),
                                       (w6_ref, b6_ref, 3, 128, 128)):
        hout = h_in - K + 1
        hp = hout // 3

        def lx_dot(xi, w_ref=w_ref, K=K, Cin=Cin, hout=hout):
            acc = None
            k = 0
            while k < K:
                nt = min(256 // Cin, K - k)
                lhs = xi[k:k + hout]
                if nt > 1:
                    lhs = jnp.concatenate(
                        [xi[k + t:k + t + hout] for t in range(nt)], axis=1)
                d = jnp.dot(lhs, w_ref[Cin * k:Cin * (k + nt), :],
                            preferred_element_type=jnp.float32)
                acc = d if acc is None else acc + d
                k += nt
            return acc

        islot = iter(range(_B))

        def lx_pool(acc, b_ref=b_ref, Cout=Cout, hout=hout, hp=hp):
            i = next(islot)
            pbuf[i, pl.ds(0, hout), pl.ds(0, Cout)] = (
                jnp.maximum(acc + b_ref[...], 0.0))
            p0 = pbuf[i, pl.ds(0, hp, stride=3), pl.ds(0, Cout)]
            p1 = pbuf[i, pl.ds(1, hp, stride=3), pl.ds(0, Cout)]
            p2 = pbuf[i, pl.ds(2, hp, stride=3), pl.ds(0, Cout)]
            return jnp.maximum(jnp.maximum(p0, p1), p2).astype(jnp.bfloat16)

        xin = staggered(xin, lx_dot, lx_pool)
        h_in = hp

    f0 = jnp.concatenate([xin[i][0:1] for i in range(_B)], axis=0)
    f1 = jnp.concatenate([xin[i][1:2] for i in range(_B)], axis=0)
    y1 = (jnp.dot(f0, nw1_ref[0], preferred_element_type=jnp.float32)
          + jnp.dot(f1, nw1_ref[1], preferred_element_type=jnp.float32)
          + nb1_ref[...])
    y1 = jnp.maximum(y1, 0.0).astype(jnp.bfloat16)
    y2 = jnp.dot(y1, nw2_ref[...], preferred_element_type=jnp.float32)
    out_ref[:, 0, :] = jnp.maximum(y2 + nb2_ref[...], 0.0).astype(out_ref.dtype)


def _bilstm_head_kernel(feat_ref, wih_ref, bg_ref, whh_ref,
                        w3f_ref, w3b_ref, b3_ref, w4_ref, b4_ref,
                        out_ref, xg_ref):
    T = feat_ref.shape[0]
    Hd = whh_ref.shape[0]
    G4 = 4 * Hd

    xg_ref[...] = (jnp.dot(feat_ref[...], wih_ref[...],
                           preferred_element_type=jnp.float32) + bg_ref[...])

    def cell(s, carry):
        h, c = carry
        hm = jnp.dot(h.astype(jnp.bfloat16), whh_ref[...],
                     preferred_element_type=jnp.float32)
        gf = xg_ref[pl.ds(s, 1), pl.ds(0, G4)] + hm[0:1, 0:G4]
        gr = xg_ref[pl.ds(T - 1 - s, 1), pl.ds(G4, G4)] + hm[1:2, G4:2 * G4]
        g = jnp.concatenate([gf, gr], axis=0)
        i_g = jax.nn.sigmoid(g[:, 0:Hd])
        f_g = jax.nn.sigmoid(g[:, Hd:2 * Hd])
        g_g = jnp.tanh(g[:, 2 * Hd:3 * Hd])
        o_g = jax.nn.sigmoid(g[:, 3 * Hd:4 * Hd])
        c = f_g * c + i_g * g_g
        h = o_g * jnp.tanh(c)
        return h, c

    def step8(q, carry):
        for u in range(8):
            carry = cell(8 * q + u, carry)
        return carry

    z = jnp.zeros((2, Hd), jnp.float32)
    h, _ = jax.lax.fori_loop(0, T // 8, step8, (z, z))

    y3 = (jnp.dot(h[0:1].astype(jnp.bfloat16), w3f_ref[...],
                  preferred_element_type=jnp.float32)
          + jnp.dot(h[1:2].astype(jnp.bfloat16), w3b_ref[...],
                    preferred_element_type=jnp.float32)
          + b3_ref[...])
    out_ref[...] = jnp.dot(y3.astype(jnp.bfloat16), w4_ref[...],
                           preferred_element_type=jnp.float32) + b4_ref[...]


def kernel(x, conv1_w, conv1_b, conv2_w, conv2_b, conv3_w, conv3_b,
           conv4_w, conv4_b, conv5_w, conv5_b, conv6_w, conv6_b,
           n1_w1, n1_b1, n1_w2, n1_b2,
           lstm_wih, lstm_bg, lstm_whh_f, lstm_whh_r,
           n3_w1f, n3_w1b, n3_b1, n3_w2, n3_b2):
    N, C, H, _ = x.shape
    assert (C, H) == (12, 3000), "conv schedule is pinned to C=12, H=3000"

    x_nhc = jnp.transpose(x[..., 0], (0, 2, 1)).astype(jnp.bfloat16)
    SR = 334
    xr = jnp.pad(x_nhc, ((0, 0), (0, 9 * SR - H), (0, 0))).reshape(
        N, SR, 9 * C)

    w1p = _poly_w(conv1_w, 9, 5)
    b1p = jnp.tile(conv1_b, (1, 9))
    w2p = _poly_w(conv2_w, 3, 4)
    b2p = jnp.tile(conv2_b, (1, 3))
    w3 = conv3_w.reshape(-1, conv3_w.shape[2])
    w4 = conv4_w.reshape(-1, conv4_w.shape[2])
    w5 = conv5_w.reshape(-1, conv5_w.shape[2])
    w6 = conv6_w.reshape(-1, conv6_w.shape[2])

    inputs = [xr, w1p, b1p, w2p, b2p, w3, conv3_b, w4, conv4_b, w5, conv5_b,
              w6, conv6_b, n1_w1, n1_b1, n1_w2, n1_b2]
    in_specs = [pl.BlockSpec((_B, SR, 9 * C), lambda n: (n, 0, 0))]
    for a in inputs[1:]:
        in_specs.append(
            pl.BlockSpec(a.shape, lambda n, nd=a.ndim: (0,) * nd))

    feats = pl.pallas_call(
        _conv_feats_kernel,
        out_shape=jax.ShapeDtypeStruct((N, 1, 128), jnp.bfloat16),
        grid_spec=pltpu.PrefetchScalarGridSpec(
            num_scalar_prefetch=0,
            grid=(N // _B,),
            in_specs=in_specs,
            out_specs=pl.BlockSpec((_B, 1, 128), lambda n: (n, 0, 0)),
            scratch_shapes=[pltpu.VMEM((_B, 320, 128), jnp.float32)],
        ),
        compiler_params=pltpu.CompilerParams(
            dimension_semantics=("parallel",),
            vmem_limit_bytes=64 * 1024 * 1024,
        ),
    )(*inputs)

    whh_st = jnp.concatenate([lstm_whh_f, lstm_whh_r], axis=1)
    return pl.pallas_call(
        _bilstm_head_kernel,
        out_shape=jax.ShapeDtypeStruct((1, n3_b2.shape[1]), jnp.float32),
        scratch_shapes=[pltpu.VMEM((N, 2048), jnp.float32)],
    )(feats.reshape(N, 128), lstm_wih, lstm_bg, whh_st,
      n3_w1f, n3_w1b, n3_b1, n3_w2, n3_b2)


# speedup vs baseline: 1.3182x; 1.0327x over previous
import jax
import jax.numpy as jnp
from jax.experimental import pallas as pl
from jax.experimental.pallas import tpu as pltpu


def _poly_w(w, G, J):
    K, Cin, Cout = w.shape
    j = jnp.arange(J)[:, None, None]
    g = jnp.arange(G)[None, :, None]
    r = jnp.arange(G)[None, None, :]
    k = G * j + g - r
    valid = (k >= 0) & (k < K)
    wk = jnp.where(valid[..., None, None], w[jnp.clip(k, 0, K - 1)], 0)
    return jnp.transpose(wk, (0, 1, 3, 2, 4)).reshape(J * G * Cin, G * Cout)


_B = 16


def _conv_feats_kernel(x_ref, w1_ref, b1_ref, w2_ref, b2_ref, w3_ref, b3_ref,
                       w4_ref, b4_ref, w5_ref, b5_ref, w6_ref, b6_ref,
                       nw1_ref, nb1_ref, nw2_ref, nb2_ref, out_ref, pbuf):

    def staggered(items, emit_dot, emit_pool):
        outs, prev = [], None
        for it in items:
            acc = emit_dot(it)
            if prev is not None:
                outs.append(emit_pool(prev))
            prev = acc
        outs.append(emit_pool(prev))
        return outs

    def l1_dot(i):
        acc = None
        for j, nt in ((0, 2), (2, 2), (4, 1)):
            lhs = x_ref[i, j:j + 330, :]
            if nt == 2:
                lhs = jnp.concatenate([lhs, x_ref[i, j + 1:j + 331, :]],
                                      axis=1)
            d = jnp.dot(lhs, w1_ref[108 * j:108 * (j + nt), :],
                        preferred_element_type=jnp.float32)
            acc = d if acc is None else acc + d
        return acc

    def l1_pool(acc):
        yi = jnp.maximum(acc + b1_ref[...], 0.0).astype(jnp.bfloat16)
        return jnp.concatenate(
            [jnp.maximum(jnp.maximum(yi[:, 96 * v:96 * v + 32],
                                     yi[:, 96 * v + 32:96 * v + 64]),
                         yi[:, 96 * v + 64:96 * v + 96]) for v in range(3)],
            axis=1)

    pooled1 = staggered(range(_B), l1_dot, l1_pool)

    def l2_dot(p1):
        acc = None
        for j in (0, 2):
            lhs = jnp.concatenate([p1[j:j + 327], p1[j + 1:j + 328]], axis=1)
            d = jnp.dot(lhs, w2_ref[96 * j:96 * (j + 2), :],
                        preferred_element_type=jnp.float32)
            acc = d if acc is None else acc + d
        return acc

    def l2_pool(acc):
        yi = jnp.maximum(acc + b2_ref[...], 0.0).astype(jnp.bfloat16)
        return jnp.maximum(jnp.maximum(yi[:, 0:64], yi[:, 64:128]),
                           yi[:, 128:192])

    xin = staggered(pooled1, l2_dot, l2_pool)

    h_in = 327
    for w_ref, b_ref, K, Cin, Cout in ((w3_ref, b3_ref, 10, 64, 64),
                                       (w4_ref, b4_ref, 5, 64, 64),
                                       (w5_ref, b5_ref, 5, 64, 128),
                                       (w6_ref, b6_ref, 3, 128, 128)):
        hout = h_in - K + 1
        hp = hout // 3

        def lx_dot(xi, w_ref=w_ref, K=K, Cin=Cin, hout=hout):
            acc = None
            k = 0
            while k < K:
                nt = min(256 // Cin, K - k)
                lhs = xi[k:k + hout]
                if nt > 1:
                    lhs = jnp.concatenate(
                        [xi[k + t:k + t + hout] for t in range(nt)], axis=1)
                d = jnp.dot(lhs, w_ref[Cin * k:Cin * (k + nt), :],
                            preferred_element_type=jnp.float32)
                acc = d if acc is None else acc + d
                k += nt
            return acc

        islot = iter(range(_B))

        def lx_pool(acc, b_ref=b_ref, Cout=Cout, hout=hout, hp=hp):
            i = next(islot)
            pbuf[i, pl.ds(0, hout), pl.ds(0, Cout)] = (
                jnp.maximum(acc + b_ref[...], 0.0))
            p0 = pbuf[i, pl.ds(0, hp, stride=3), pl.ds(0, Cout)]
            p1 = pbuf[i, pl.ds(1, hp, stride=3), pl.ds(0, Cout)]
            p2 = pbuf[i, pl.ds(2, hp, stride=3), pl.ds(0, Cout)]
            return jnp.maximum(jnp.maximum(p0, p1), p2).astype(jnp.bfloat16)

        xin = staggered(xin, lx_dot, lx_pool)
        h_in = hp

    f0 = jnp.concatenate([xin[i][0:1] for i in range(_B)], axis=0)
    f1 = jnp.concatenate([xin[i][1:2] for i in range(_B)], axis=0)
    y1 = (jnp.dot(f0, nw1_ref[0], preferred_element_type=jnp.float32)
          + jnp.dot(f1, nw1_ref[1], preferred_element_type=jnp.float32)
          + nb1_ref[...])
    y1 = jnp.maximum(y1, 0.0).astype(jnp.bfloat16)
    y2 = jnp.dot(y1, nw2_ref[...], preferred_element_type=jnp.float32)
    out_ref[:, 0, :] = jnp.maximum(y2 + nb2_ref[...], 0.0).astype(out_ref.dtype)


def _bilstm_head_kernel(feat_ref, wih_ref, bg_ref, whh_ref,
                        w3f_ref, w3b_ref, b3_ref, w4_ref, b4_ref,
                        out_ref, xg_ref):
    T = feat_ref.shape[0]
    Hd = whh_ref.shape[0]
    G4 = 4 * Hd

    xg_ref[...] = (jnp.dot(feat_ref[...], wih_ref[...],
                           preferred_element_type=jnp.float32) + bg_ref[...])

    def cell(s, carry):
        h, c = carry
        hm = jnp.dot(h.astype(jnp.bfloat16), whh_ref[...],
                     preferred_element_type=jnp.float32)
        gf = xg_ref[pl.ds(s, 1), pl.ds(0, G4)] + hm[0:1, 0:G4]
        gr = xg_ref[pl.ds(T - 1 - s, 1), pl.ds(G4, G4)] + hm[1:2, G4:2 * G4]
        g = jnp.concatenate([gf, gr], axis=0)
        i_g = jax.nn.sigmoid(g[:, 0:Hd])
        f_g = jax.nn.sigmoid(g[:, Hd:2 * Hd])
        g_g = jnp.tanh(g[:, 2 * Hd:3 * Hd])
        o_g = jax.nn.sigmoid(g[:, 3 * Hd:4 * Hd])
        c = f_g * c + i_g * g_g
        h = o_g * jnp.tanh(c)
        return h, c

    def step8(q, carry):
        for u in range(8):
            carry = cell(8 * q + u, carry)
        return carry

    z = jnp.zeros((2, Hd), jnp.float32)
    h, _ = jax.lax.fori_loop(0, T // 8, step8, (z, z))

    y3 = (jnp.dot(h[0:1].astype(jnp.bfloat16), w3f_ref[...],
                  preferred_element_type=jnp.float32)
          + jnp.dot(h[1:2].astype(jnp.bfloat16), w3b_ref[...],
                    preferred_element_type=jnp.float32)
          + b3_ref[...])
    out_ref[...] = jnp.dot(y3.astype(jnp.bfloat16), w4_ref[...],
                           preferred_element_type=jnp.float32) + b4_ref[...]


def kernel(x, conv1_w, conv1_b, conv2_w, conv2_b, conv3_w, conv3_b,
           conv4_w, conv4_b, conv5_w, conv5_b, conv6_w, conv6_b,
           n1_w1, n1_b1, n1_w2, n1_b2,
           lstm_wih, lstm_bg, lstm_whh_f, lstm_whh_r,
           n3_w1f, n3_w1b, n3_b1, n3_w2, n3_b2):
    N, C, H, _ = x.shape
    assert (C, H) == (12, 3000), "conv schedule is pinned to C=12, H=3000"

    x_nhc = jnp.transpose(x[..., 0], (0, 2, 1)).astype(jnp.bfloat16)
    SR = 334
    xr = jnp.pad(x_nhc, ((0, 0), (0, 9 * SR - H), (0, 0))).reshape(
        N, SR, 9 * C)

    w1p = _poly_w(conv1_w, 9, 5)
    b1p = jnp.tile(conv1_b, (1, 9))
    w2p = _poly_w(conv2_w, 3, 4)
    b2p = jnp.tile(conv2_b, (1, 3))
    w3 = conv3_w.reshape(-1, conv3_w.shape[2])
    w4 = conv4_w.reshape(-1, conv4_w.shape[2])
    w5 = conv5_w.reshape(-1, conv5_w.shape[2])
    w6 = conv6_w.reshape(-1, conv6_w.shape[2])

    inputs = [xr, w1p, b1p, w2p, b2p, w3, conv3_b, w4, conv4_b, w5, conv5_b,
              w6, conv6_b, n1_w1, n1_b1, n1_w2, n1_b2]
    in_specs = [pl.BlockSpec((_B, SR, 9 * C), lambda n: (n, 0, 0))]
    for a in inputs[1:]:
        in_specs.append(
            pl.BlockSpec(a.shape, lambda n, nd=a.ndim: (0,) * nd))

    feats = pl.pallas_call(
        _conv_feats_kernel,
        out_shape=jax.ShapeDtypeStruct((N, 1, 128), jnp.bfloat16),
        grid_spec=pltpu.PrefetchScalarGridSpec(
            num_scalar_prefetch=0,
            grid=(N // _B,),
            in_specs=in_specs,
            out_specs=pl.BlockSpec((_B, 1, 128), lambda n: (n, 0, 0)),
            scratch_shapes=[pltpu.VMEM((_B, 320, 128), jnp.float32)],
        ),
        compiler_params=pltpu.CompilerParams(
            dimension_semantics=("parallel",),
            vmem_limit_bytes=64 * 1024 * 1024,
        ),
    )(*inputs)

    whh_st = jnp.concatenate([lstm_whh_f, lstm_whh_r], axis=1)
    return pl.pallas_call(
        _bilstm_head_kernel,
        out_shape=jax.ShapeDtypeStruct((1, n3_b2.shape[1]), jnp.float32),
        scratch_shapes=[pltpu.VMEM((N, 2048), jnp.float32)],
    )(feats.reshape(N, 128), lstm_wih, lstm_bg, whh_st,
      n3_w1f, n3_w1b, n3_b1, n3_w2, n3_b2)


# speedup vs baseline: 1.3245x; 1.0048x over previous
import jax
import jax.numpy as jnp
from jax.experimental import pallas as pl
from jax.experimental.pallas import tpu as pltpu


def _poly_w(w, G, J):
    K, Cin, Cout = w.shape
    j = jnp.arange(J)[:, None, None]
    g = jnp.arange(G)[None, :, None]
    r = jnp.arange(G)[None, None, :]
    k = G * j + g - r
    valid = (k >= 0) & (k < K)
    wk = jnp.where(valid[..., None, None], w[jnp.clip(k, 0, K - 1)], 0)
    return jnp.transpose(wk, (0, 1, 3, 2, 4)).reshape(J * G * Cin, G * Cout)


_B = 32


def _conv_feats_kernel(x_ref, w1_ref, b1_ref, w2_ref, b2_ref, w3_ref, b3_ref,
                       w4_ref, b4_ref, w5_ref, b5_ref, w6_ref, b6_ref,
                       nw1_ref, nb1_ref, nw2_ref, nb2_ref, out_ref, pbuf):

    def staggered(items, emit_dot, emit_pool):
        outs, prev = [], None
        for it in items:
            acc = emit_dot(it)
            if prev is not None:
                outs.append(emit_pool(prev))
            prev = acc
        outs.append(emit_pool(prev))
        return outs

    def l1_dot(i):
        acc = None
        for j, nt in ((0, 2), (2, 2), (4, 1)):
            lhs = x_ref[i, j:j + 330, :]
            if nt == 2:
                lhs = jnp.concatenate([lhs, x_ref[i, j + 1:j + 331, :]],
                                      axis=1)
            d = jnp.dot(lhs, w1_ref[108 * j:108 * (j + nt), :],
                        preferred_element_type=jnp.float32)
            acc = d if acc is None else acc + d
        return acc

    def l1_pool(acc):
        yi = jnp.maximum(acc + b1_ref[...], 0.0).astype(jnp.bfloat16)
        return jnp.concatenate(
            [jnp.maximum(jnp.maximum(yi[:, 96 * v:96 * v + 32],
                                     yi[:, 96 * v + 32:96 * v + 64]),
                         yi[:, 96 * v + 64:96 * v + 96]) for v in range(3)],
            axis=1)

    pooled1 = staggered(range(_B), l1_dot, l1_pool)

    def l2_dot(p1):
        acc = None
        for j in (0, 2):
            lhs = jnp.concatenate([p1[j:j + 327], p1[j + 1:j + 328]], axis=1)
            d = jnp.dot(lhs, w2_ref[96 * j:96 * (j + 2), :],
                        preferred_element_type=jnp.float32)
            acc = d if acc is None else acc + d
        return acc

    def l2_pool(acc):
        yi = jnp.maximum(acc + b2_ref[...], 0.0).astype(jnp.bfloat16)
        return jnp.maximum(jnp.maximum(yi[:, 0:64], yi[:, 64:128]),
                           yi[:, 128:192])

    xin = staggered(pooled1, l2_dot, l2_pool)

    h_in = 327
    for w_ref, b_ref, K, Cin, Cout in ((w3_ref, b3_ref, 10, 64, 64),
                                       (w4_ref, b4_ref, 5, 64, 64),
                                       (w5_ref, b5_ref, 5, 64, 128),
                                       (w6_ref, b6_ref, 3, 128, 128)):
        hout = h_in - K + 1
        hp = hout // 3

        def lx_dot(xi, w_ref=w_ref, K=K, Cin=Cin, hout=hout):
            acc = None
            k = 0
            while k < K:
                nt = min(256 // Cin, K - k)
                lhs = xi[k:k + hout]
                if nt > 1:
                    lhs = jnp.concatenate(
                        [xi[k + t:k + t + hout] for t in range(nt)], axis=1)
                d = jnp.dot(lhs, w_ref[Cin * k:Cin * (k + nt), :],
                            preferred_element_type=jnp.float32)
                acc = d if acc is None else acc + d
                k += nt
            return acc

        islot = iter(range(_B))

        def lx_pool(acc, b_ref=b_ref, Cout=Cout, hout=hout, hp=hp):
            i = next(islot)
            pbuf[i, pl.ds(0, hout), pl.ds(0, Cout)] = (
                jnp.maximum(acc + b_ref[...], 0.0))
            p0 = pbuf[i, pl.ds(0, hp, stride=3), pl.ds(0, Cout)]
            p1 = pbuf[i, pl.ds(1, hp, stride=3), pl.ds(0, Cout)]
            p2 = pbuf[i, pl.ds(2, hp, stride=3), pl.ds(0, Cout)]
            return jnp.maximum(jnp.maximum(p0, p1), p2).astype(jnp.bfloat16)

        xin = staggered(xin, lx_dot, lx_pool)
        h_in = hp

    f0 = jnp.concatenate([xin[i][0:1] for i in range(_B)], axis=0)
    f1 = jnp.concatenate([xin[i][1:2] for i in range(_B)], axis=0)
    y1 = (jnp.dot(f0, nw1_ref[0], preferred_element_type=jnp.float32)
          + jnp.dot(f1, nw1_ref[1], preferred_element_type=jnp.float32)
          + nb1_ref[...])
    y1 = jnp.maximum(y1, 0.0).astype(jnp.bfloat16)
    y2 = jnp.dot(y1, nw2_ref[...], preferred_element_type=jnp.float32)
    out_ref[:, 0, :] = jnp.maximum(y2 + nb2_ref[...], 0.0).astype(out_ref.dtype)


def _bilstm_head_kernel(feat_ref, wih_ref, bg_ref, whh_ref,
                        w3f_ref, w3b_ref, b3_ref, w4_ref, b4_ref,
                        out_ref, xg_ref):
    T = feat_ref.shape[0]
    Hd = whh_ref.shape[0]
    G4 = 4 * Hd

    xg_ref[...] = (jnp.dot(feat_ref[...], wih_ref[...],
                           preferred_element_type=jnp.float32) + bg_ref[...])

    def cell(s, carry):
        h, c = carry
        hm = jnp.dot(h.astype(jnp.bfloat16), whh_ref[...],
                     preferred_element_type=jnp.float32)
        gf = xg_ref[pl.ds(s, 1), pl.ds(0, G4)] + hm[0:1, 0:G4]
        gr = xg_ref[pl.ds(T - 1 - s, 1), pl.ds(G4, G4)] + hm[1:2, G4:2 * G4]
        g = jnp.concatenate([gf, gr], axis=0)
        i_g = jax.nn.sigmoid(g[:, 0:Hd])
        f_g = jax.nn.sigmoid(g[:, Hd:2 * Hd])
        g_g = jnp.tanh(g[:, 2 * Hd:3 * Hd])
        o_g = jax.nn.sigmoid(g[:, 3 * Hd:4 * Hd])
        c = f_g * c + i_g * g_g
        h = o_g * jnp.tanh(c)
        return h, c

    def step8(q, carry):
        for u in range(8):
            carry = cell(8 * q + u, carry)
        return carry

    z = jnp.zeros((2, Hd), jnp.float32)
    h, _ = jax.lax.fori_loop(0, T // 8, step8, (z, z))

    y3 = (jnp.dot(h[0:1].astype(jnp.bfloat16), w3f_ref[...],
                  preferred_element_type=jnp.float32)
          + jnp.dot(h[1:2].astype(jnp.bfloat16), w3b_ref[...],
                    preferred_element_type=jnp.float32)
          + b3_ref[...])
    out_ref[...] = jnp.dot(y3.astype(jnp.bfloat16), w4_ref[...],
                           preferred_element_type=jnp.float32) + b4_ref[...]


def kernel(x, conv1_w, conv1_b, conv2_w, conv2_b, conv3_w, conv3_b,
           conv4_w, conv4_b, conv5_w, conv5_b, conv6_w, conv6_b,
           n1_w1, n1_b1, n1_w2, n1_b2,
           lstm_wih, lstm_bg, lstm_whh_f, lstm_whh_r,
           n3_w1f, n3_w1b, n3_b1, n3_w2, n3_b2):
    N, C, H, _ = x.shape
    assert (C, H) == (12, 3000), "conv schedule is pinned to C=12, H=3000"

    x_nhc = jnp.transpose(x[..., 0], (0, 2, 1)).astype(jnp.bfloat16)
    SR = 334
    xr = jnp.pad(x_nhc, ((0, 0), (0, 9 * SR - H), (0, 0))).reshape(
        N, SR, 9 * C)

    w1p = _poly_w(conv1_w, 9, 5)
    b1p = jnp.tile(conv1_b, (1, 9))
    w2p = _poly_w(conv2_w, 3, 4)
    b2p = jnp.tile(conv2_b, (1, 3))
    w3 = conv3_w.reshape(-1, conv3_w.shape[2])
    w4 = conv4_w.reshape(-1, conv4_w.shape[2])
    w5 = conv5_w.reshape(-1, conv5_w.shape[2])
    w6 = conv6_w.reshape(-1, conv6_w.shape[2])

    inputs = [xr, w1p, b1p, w2p, b2p, w3, conv3_b, w4, conv4_b, w5, conv5_b,
              w6, conv6_b, n1_w1, n1_b1, n1_w2, n1_b2]
    in_specs = [pl.BlockSpec((_B, SR, 9 * C), lambda n: (n, 0, 0))]
    for a in inputs[1:]:
        in_specs.append(
            pl.BlockSpec(a.shape, lambda n, nd=a.ndim: (0,) * nd))

    feats = pl.pallas_call(
        _conv_feats_kernel,
        out_shape=jax.ShapeDtypeStruct((N, 1, 128), jnp.bfloat16),
        grid_spec=pltpu.PrefetchScalarGridSpec(
            num_scalar_prefetch=0,
            grid=(N // _B,),
            in_specs=in_specs,
            out_specs=pl.BlockSpec((_B, 1, 128), lambda n: (n, 0, 0)),
            scratch_shapes=[pltpu.VMEM((_B, 320, 128), jnp.float32)],
        ),
        compiler_params=pltpu.CompilerParams(
            dimension_semantics=("parallel",),
            vmem_limit_bytes=64 * 1024 * 1024,
        ),
    )(*inputs)

    whh_st = jnp.concatenate([lstm_whh_f, lstm_whh_r], axis=1)
    return pl.pallas_call(
        _bilstm_head_kernel,
        out_shape=jax.ShapeDtypeStruct((1, n3_b2.shape[1]), jnp.float32),
        scratch_shapes=[pltpu.VMEM((N, 2048), jnp.float32)],
    )(feats.reshape(N, 128), lstm_wih, lstm_bg, whh_st,
      n3_w1f, n3_w1b, n3_b1, n3_w2, n3_b2)
